# baseline (device time: 52651 ns/iter reference)
import jax
import jax.numpy as jnp
from jax import lax
from jax.experimental import pallas as pl
from jax.experimental.pallas import tpu as pltpu

N_Q = 4
C = 8


def kernel(partial, resid, gamma):
    m, d = resid.shape
    qrows = m // N_Q
    csz = qrows // C

    def body(partial_ref, resid_ref, gamma_ref, out_ref,
             p_loc, res_loc, p_bf, recv_x, gather_bf,
             x_s, x_r, yo_s, yo_r, zo_s, zo_r, yf_s, yf_r, zf_s, zf_r,
             loc_sem):
        my_x = lax.axis_index("x")
        my_y = lax.axis_index("y")
        my_z = lax.axis_index("z")
        x_peer = (1 - my_x, my_y, my_z)
        y_peer = (my_x, 1 - my_y, my_z)
        z_peer = (my_x, my_y, 1 - my_z)

        q = 2 * my_y + my_z
        qy = 2 * (1 - my_y) + my_z
        qz = 2 * my_y + (1 - my_z)
        row_q = q * qrows
        row_qy = qy * qrows
        row_qz = qz * qrows

        cp_p = pltpu.make_async_copy(
            partial_ref.at[0, pl.ds(row_q, qrows), :], p_loc, loc_sem.at[0])
        cp_r = pltpu.make_async_copy(
            resid_ref.at[pl.ds(row_q, qrows), :], res_loc, loc_sem.at[1])
        cp_p.start()
        cp_r.start()

        barrier = pltpu.get_barrier_semaphore()
        for nbr in (x_peer, y_peer, z_peer):
            pl.semaphore_signal(barrier, inc=1, device_id=nbr,
                                device_id_type=pl.DeviceIdType.MESH)
        pl.semaphore_wait(barrier, 3)

        cp_p.wait()
        p_bf[...] = p_loc[...].astype(jnp.bfloat16)

        x_ops = []
        for c in range(C):
            op = pltpu.make_async_remote_copy(
                src_ref=p_bf.at[pl.ds(c * csz, csz), :],
                dst_ref=recv_x.at[pl.ds(c * csz, csz), :],
                send_sem=x_s.at[c], recv_sem=x_r.at[c],
                device_id=x_peer, device_id_type=pl.DeviceIdType.MESH)
            op.start()
            x_ops.append(op)

        cp_r.wait()
        gam = gamma_ref[...]

        y_own, z_own = [], []
        for c in range(C):
            x_ops[c].wait_recv()
            sl = pl.ds(c * csz, csz)
            y = p_loc[sl, :] + recv_x[sl, :].astype(jnp.float32) \
                + res_loc[sl, :]
            r = lax.rsqrt(jnp.mean(y * y, axis=-1, keepdims=True) + 1e-6)
            o = y * r * gam
            gsl = pl.ds(row_q + c * csz, csz)
            out_ref[gsl, :] = o
            gather_bf[gsl, :] = o.astype(jnp.bfloat16)
            for peer, s_arr, r_arr, lst in (
                    (y_peer, yo_s, yo_r, y_own),
                    (z_peer, zo_s, zo_r, z_own)):
                op = pltpu.make_async_remote_copy(
                    src_ref=gather_bf.at[gsl, :],
                    dst_ref=gather_bf.at[gsl, :],
                    send_sem=s_arr.at[c], recv_sem=r_arr.at[c],
                    device_id=peer, device_id_type=pl.DeviceIdType.MESH)
                op.start()
                lst.append(op)

        def upcast(row):
            sl = pl.ds(row, csz)
            out_ref[sl, :] = gather_bf[sl, :].astype(jnp.float32)

        y_fwd = []
        for c in range(C // 2):
            z_own[c].wait_recv()
            op = pltpu.make_async_remote_copy(
                src_ref=gather_bf.at[pl.ds(row_qz + c * csz, csz), :],
                dst_ref=gather_bf.at[pl.ds(row_qz + c * csz, csz), :],
                send_sem=yf_s.at[c], recv_sem=yf_r.at[c],
                device_id=y_peer, device_id_type=pl.DeviceIdType.MESH)
            op.start()
            y_fwd.append(op)
            upcast(row_qz + c * csz)

        z_fwd = []
        for c in range(C // 2, C):
            y_own[c].wait_recv()
            op = pltpu.make_async_remote_copy(
                src_ref=gather_bf.at[pl.ds(row_qy + c * csz, csz), :],
                dst_ref=gather_bf.at[pl.ds(row_qy + c * csz, csz), :],
                send_sem=zf_s.at[c - C // 2], recv_sem=zf_r.at[c - C // 2],
                device_id=z_peer, device_id_type=pl.DeviceIdType.MESH)
            op.start()
            z_fwd.append(op)
            upcast(row_qy + c * csz)

        for c in range(C // 2, C):
            z_own[c].wait_recv()
            upcast(row_qz + c * csz)
        for c in range(C // 2):
            y_own[c].wait_recv()
            upcast(row_qy + c * csz)
        row_qd = (2 * (1 - my_y) + (1 - my_z)) * qrows
        for c, op in enumerate(y_fwd):
            op.wait_recv()
            upcast(row_qd + c * csz)
        for c, op in enumerate(z_fwd):
            op.wait_recv()
            upcast(row_qd + (C // 2 + c) * csz)
        for op in x_ops + y_own + z_own + y_fwd + z_fwd:
            op.wait_send()

    return pl.pallas_call(
        body,
        out_shape=jax.ShapeDtypeStruct((m, d), jnp.float32),
        in_specs=[
            pl.BlockSpec(memory_space=pl.ANY),
            pl.BlockSpec(memory_space=pl.ANY),
            pl.BlockSpec(memory_space=pltpu.VMEM),
        ],
        out_specs=pl.BlockSpec(memory_space=pltpu.VMEM),
        scratch_shapes=[
            pltpu.VMEM((qrows, d), jnp.float32),
            pltpu.VMEM((qrows, d), jnp.float32),
            pltpu.VMEM((qrows, d), jnp.bfloat16),
            pltpu.VMEM((qrows, d), jnp.bfloat16),
            pltpu.VMEM((m, d), jnp.bfloat16),
            pltpu.SemaphoreType.DMA((C,)),
            pltpu.SemaphoreType.DMA((C,)),
            pltpu.SemaphoreType.DMA((C,)),
            pltpu.SemaphoreType.DMA((C,)),
            pltpu.SemaphoreType.DMA((C,)),
            pltpu.SemaphoreType.DMA((C,)),
            pltpu.SemaphoreType.DMA((C // 2,)),
            pltpu.SemaphoreType.DMA((C // 2,)),
            pltpu.SemaphoreType.DMA((C // 2,)),
            pltpu.SemaphoreType.DMA((C // 2,)),
            pltpu.SemaphoreType.DMA((2,)),
        ],
        compiler_params=pltpu.CompilerParams(
            collective_id=0, has_side_effects=True
        ),
    )(partial, resid, gamma.reshape(1, d))


# device time: 50423 ns/iter; 1.0442x vs baseline; 1.0442x over previous
import jax
import jax.numpy as jnp
from jax import lax
from jax.experimental import pallas as pl
from jax.experimental.pallas import tpu as pltpu

N_Q = 4
C = 16


def kernel(partial, resid, gamma):
    m, d = resid.shape
    qrows = m // N_Q
    csz = qrows // C

    def body(partial_ref, resid_ref, gamma_ref, out_ref,
             p_loc, res_loc, p_bf, recv_x, gather_bf,
             x_s, x_r, yo_s, yo_r, zo_s, zo_r, yf_s, yf_r, zf_s, zf_r,
             ploc_sem, loc_sem):
        my_x = lax.axis_index("x")
        my_y = lax.axis_index("y")
        my_z = lax.axis_index("z")
        x_peer = (1 - my_x, my_y, my_z)
        y_peer = (my_x, 1 - my_y, my_z)
        z_peer = (my_x, my_y, 1 - my_z)

        q = 2 * my_y + my_z
        qy = 2 * (1 - my_y) + my_z
        qz = 2 * my_y + (1 - my_z)
        row_q = q * qrows
        row_qy = qy * qrows
        row_qz = qz * qrows

        cp_ps = []
        for c in range(C):
            sl = pl.ds(c * csz, csz)
            cp = pltpu.make_async_copy(
                partial_ref.at[0, pl.ds(row_q + c * csz, csz), :],
                p_loc.at[sl, :], ploc_sem.at[c])
            cp.start()
            cp_ps.append(cp)
        cp_r = pltpu.make_async_copy(
            resid_ref.at[pl.ds(row_q, qrows), :], res_loc, loc_sem.at[0])
        cp_r.start()

        barrier = pltpu.get_barrier_semaphore()
        for nbr in (x_peer, y_peer, z_peer):
            pl.semaphore_signal(barrier, inc=1, device_id=nbr,
                                device_id_type=pl.DeviceIdType.MESH)
        pl.semaphore_wait(barrier, 3)

        x_ops = []
        for c in range(C):
            cp_ps[c].wait()
            sl = pl.ds(c * csz, csz)
            p_bf[sl, :] = p_loc[sl, :].astype(jnp.bfloat16)
            op = pltpu.make_async_remote_copy(
                src_ref=p_bf.at[pl.ds(c * csz, csz), :],
                dst_ref=recv_x.at[pl.ds(c * csz, csz), :],
                send_sem=x_s.at[c], recv_sem=x_r.at[c],
                device_id=x_peer, device_id_type=pl.DeviceIdType.MESH)
            op.start()
            x_ops.append(op)

        cp_r.wait()
        gam = gamma_ref[...]

        y_own, z_own = [], []
        for c in range(C):
            x_ops[c].wait_recv()
            sl = pl.ds(c * csz, csz)
            y = p_loc[sl, :] + recv_x[sl, :].astype(jnp.float32) \
                + res_loc[sl, :]
            r = lax.rsqrt(jnp.mean(y * y, axis=-1, keepdims=True) + 1e-6)
            o = y * r * gam
            gsl = pl.ds(row_q + c * csz, csz)
            gather_bf[gsl, :] = o.astype(jnp.bfloat16)
            for peer, s_arr, r_arr, lst in (
                    (y_peer, yo_s, yo_r, y_own),
                    (z_peer, zo_s, zo_r, z_own)):
                op = pltpu.make_async_remote_copy(
                    src_ref=gather_bf.at[gsl, :],
                    dst_ref=gather_bf.at[gsl, :],
                    send_sem=s_arr.at[c], recv_sem=r_arr.at[c],
                    device_id=peer, device_id_type=pl.DeviceIdType.MESH)
                op.start()
                lst.append(op)
            out_ref[gsl, :] = o

        def upcast(row):
            sl = pl.ds(row, csz)
            out_ref[sl, :] = gather_bf[sl, :].astype(jnp.float32)

        y_fwd = []
        for c in range(C // 2):
            z_own[c].wait_recv()
            op = pltpu.make_async_remote_copy(
                src_ref=gather_bf.at[pl.ds(row_qz + c * csz, csz), :],
                dst_ref=gather_bf.at[pl.ds(row_qz + c * csz, csz), :],
                send_sem=yf_s.at[c], recv_sem=yf_r.at[c],
                device_id=y_peer, device_id_type=pl.DeviceIdType.MESH)
            op.start()
            y_fwd.append(op)
            upcast(row_qz + c * csz)

        z_fwd = []
        for c in range(C // 2, C):
            y_own[c].wait_recv()
            op = pltpu.make_async_remote_copy(
                src_ref=gather_bf.at[pl.ds(row_qy + c * csz, csz), :],
                dst_ref=gather_bf.at[pl.ds(row_qy + c * csz, csz), :],
                send_sem=zf_s.at[c - C // 2], recv_sem=zf_r.at[c - C // 2],
                device_id=z_peer, device_id_type=pl.DeviceIdType.MESH)
            op.start()
            z_fwd.append(op)
            upcast(row_qy + c * csz)

        for c in range(C // 2, C):
            z_own[c].wait_recv()
            upcast(row_qz + c * csz)
        for c in range(C // 2):
            y_own[c].wait_recv()
            upcast(row_qy + c * csz)
        row_qd = (2 * (1 - my_y) + (1 - my_z)) * qrows
        for c, op in enumerate(y_fwd):
            op.wait_recv()
            upcast(row_qd + c * csz)
        for c, op in enumerate(z_fwd):
            op.wait_recv()
            upcast(row_qd + (C // 2 + c) * csz)
        for op in x_ops + y_own + z_own + y_fwd + z_fwd:
            op.wait_send()

    return pl.pallas_call(
        body,
        out_shape=jax.ShapeDtypeStruct((m, d), jnp.float32),
        in_specs=[
            pl.BlockSpec(memory_space=pl.ANY),
            pl.BlockSpec(memory_space=pl.ANY),
            pl.BlockSpec(memory_space=pltpu.VMEM),
        ],
        out_specs=pl.BlockSpec(memory_space=pltpu.VMEM),
        scratch_shapes=[
            pltpu.VMEM((qrows, d), jnp.float32),
            pltpu.VMEM((qrows, d), jnp.float32),
            pltpu.VMEM((qrows, d), jnp.bfloat16),
            pltpu.VMEM((qrows, d), jnp.bfloat16),
            pltpu.VMEM((m, d), jnp.bfloat16),
            pltpu.SemaphoreType.DMA((C,)),
            pltpu.SemaphoreType.DMA((C,)),
            pltpu.SemaphoreType.DMA((C,)),
            pltpu.SemaphoreType.DMA((C,)),
            pltpu.SemaphoreType.DMA((C,)),
            pltpu.SemaphoreType.DMA((C,)),
            pltpu.SemaphoreType.DMA((C // 2,)),
            pltpu.SemaphoreType.DMA((C // 2,)),
            pltpu.SemaphoreType.DMA((C // 2,)),
            pltpu.SemaphoreType.DMA((C // 2,)),
            pltpu.SemaphoreType.DMA((C,)),
            pltpu.SemaphoreType.DMA((1,)),
        ],
        compiler_params=pltpu.CompilerParams(
            collective_id=0, has_side_effects=True
        ),
    )(partial, resid, gamma.reshape(1, d))
